# Initial kernel scaffold; baseline (speedup 1.0000x reference)
import jax
import jax.numpy as jnp
from jax.experimental import pallas as pl


def kernel(g, h, e, W_feat, b_feat, Wl1, bl1, Wr1, br1, attn1, gamma1, beta1, Wl2, bl2, Wr2, br2, attn2, gamma2, beta2, Wm0, bm0, Wm1, bm1, Wm2, bm2):
    raise NotImplementedError("write your pallas kernel here")


# speedup vs baseline: 7.2924x; 7.2924x over previous
import functools

import jax
import jax.numpy as jnp
from jax import lax
from jax.experimental import pallas as pl
from jax.experimental.pallas import tpu as pltpu
from jax.experimental.pallas import tpu_sc as plsc

N = 10000
E = 320000
D = 128
HW = 64
AW = 80
NEG = 0.2

SC_TILES = 16
EPT = E // SC_TILES
CH = 64
NFULL = EPT // CH
TAIL = EPT - NFULL * CH
NPAD = 10240
RPT = NPAD // SC_TILES

R = 1000
GRID = N // R
EB = E // GRID

_SC_PARAMS = pltpu.CompilerParams(needs_layout_passes=False,
                                  use_tc_tiling_on_sc=False)


def _proj3_body(h_ref, wf_ref, bf_ref, wl_ref, bl_ref, wr_ref, br_ref,
                x_ref, fsa_ref, fsb_ref, fda_ref, fdb_ref):
    x = jnp.dot(h_ref[...], wf_ref[...], preferred_element_type=jnp.float32)
    x = x + bf_ref[...]
    x_ref[...] = x
    fs = jnp.dot(x, wl_ref[...], preferred_element_type=jnp.float32) + bl_ref[...]
    fd = jnp.dot(x, wr_ref[...], preferred_element_type=jnp.float32) + br_ref[...]
    fsa_ref[...] = fs[:, :HW]
    fsb_ref[...] = fs[:, HW:]
    fda_ref[...] = fd[:, :HW]
    fdb_ref[...] = fd[:, HW:]


def _proj3(h, wf, bf, wl, bl, wr, br):
    w_spec = pl.BlockSpec((D, D), lambda i: (0, 0))
    b_spec = pl.BlockSpec((1, D), lambda i: (0, 0))
    r_spec = pl.BlockSpec((R, D), lambda i: (i, 0))
    h_spec = pl.BlockSpec((R, HW), lambda i: (i, 0))
    return pl.pallas_call(
        _proj3_body,
        grid=(GRID,),
        in_specs=[r_spec, w_spec, b_spec, w_spec, b_spec, w_spec, b_spec],
        out_specs=[r_spec, h_spec, h_spec, h_spec, h_spec],
        out_shape=[jax.ShapeDtypeStruct((N, D), jnp.float32)]
        + [jax.ShapeDtypeStruct((N, HW), jnp.float32)] * 4,
    )(h, wf, bf.reshape(1, D), wl, bl.reshape(1, D), wr, br.reshape(1, D))


def _combine_body(heads, op_ref, exp_ref, o_ref, st_ref):
    i = pl.program_id(0)
    o = jnp.concatenate([op_ref[0, :, :HW], op_ref[1, :, :HW]], axis=1)
    if heads == 8:
        s8 = jnp.concatenate([op_ref[0, :, HW:HW + 4],
                              op_ref[1, :, HW:HW + 4]], axis=1)
        den = jnp.dot(s8, exp_ref[...], preferred_element_type=jnp.float32) + 1e-9
        o = o / den
    else:
        o = o / (op_ref[0, :, HW:HW + 1] + 1e-9)
    o_ref[...] = o

    @pl.when(i == 0)
    def _():
        st_ref[...] = jnp.zeros_like(st_ref)

    upd = jnp.concatenate(
        [jnp.sum(o, axis=0, keepdims=True),
         jnp.sum(o * o, axis=0, keepdims=True),
         jnp.zeros((6, D), jnp.float32)], axis=0)
    st_ref[...] = st_ref[...] + upd


def _combine(heads, out_p, expand):
    return pl.pallas_call(
        functools.partial(_combine_body, heads),
        grid=(GRID,),
        in_specs=[pl.BlockSpec((2, R, AW), lambda i: (0, i, 0)),
                  pl.BlockSpec((8, D), lambda i: (0, 0))],
        out_specs=[pl.BlockSpec((R, D), lambda i: (i, 0)),
                   pl.BlockSpec((8, D), lambda i: (0, 0))],
        out_shape=[jax.ShapeDtypeStruct((N, D), jnp.float32),
                   jax.ShapeDtypeStruct((8, D), jnp.float32)],
    )(out_p, expand)


def _bnproj_body(o_ref, st_ref, g_ref, b_ref, res_ref, wl_ref, bl_ref,
                 wr_ref, br_ref, x2_ref, fsa_ref, fsb_ref, fda_ref, fdb_ref):
    mu = st_ref[0:1, :] / N
    var = st_ref[1:2, :] / N - mu * mu
    inv = lax.rsqrt(var + 1e-5)
    xb = g_ref[...] * (o_ref[...] - mu) * inv + b_ref[...]
    el = jnp.where(xb > 0, xb, jnp.exp(xb) - 1.0)
    x2 = el + res_ref[...]
    x2_ref[...] = x2
    fs = jnp.dot(x2, wl_ref[...], preferred_element_type=jnp.float32) + bl_ref[...]
    fd = jnp.dot(x2, wr_ref[...], preferred_element_type=jnp.float32) + br_ref[...]
    fsa_ref[...] = fs[:, :HW]
    fsb_ref[...] = fs[:, HW:]
    fda_ref[...] = fd[:, :HW]
    fdb_ref[...] = fd[:, HW:]


def _bnproj(o_raw, st, gamma, beta, res, wl, bl, wr, br):
    w_spec = pl.BlockSpec((D, D), lambda i: (0, 0))
    b_spec = pl.BlockSpec((1, D), lambda i: (0, 0))
    r_spec = pl.BlockSpec((R, D), lambda i: (i, 0))
    h_spec = pl.BlockSpec((R, HW), lambda i: (i, 0))
    return pl.pallas_call(
        _bnproj_body,
        grid=(GRID,),
        in_specs=[r_spec, pl.BlockSpec((8, D), lambda i: (0, 0)),
                  b_spec, b_spec, r_spec, w_spec, b_spec, w_spec, b_spec],
        out_specs=[r_spec, h_spec, h_spec, h_spec, h_spec],
        out_shape=[jax.ShapeDtypeStruct((N, D), jnp.float32)]
        + [jax.ShapeDtypeStruct((N, HW), jnp.float32)] * 4,
    )(o_raw, st, gamma.reshape(1, D), beta.reshape(1, D), res,
      wl, bl.reshape(1, D), wr, br.reshape(1, D))


def _edge_exp_body(p_ref, ex_ref):
    ex_ref[...] = jnp.exp(p_ref[0] + p_ref[1])


def _edge_exp(plog3):
    nb = E // D
    return pl.pallas_call(
        _edge_exp_body,
        in_specs=[pl.BlockSpec((2, nb, D), lambda: (0, 0, 0))],
        out_specs=pl.BlockSpec((nb, D), lambda: (0, 0)),
        out_shape=jax.ShapeDtypeStruct((nb, D), jnp.float32),
    )(plog3)


def _bnmlp_body(o_ref, st_ref, g_ref, b_ref, res_ref, w0_ref, b0_ref,
                w1_ref, b1_ref, w2_ref, b2_ref, y_ref):
    mu = st_ref[0:1, :] / N
    var = st_ref[1:2, :] / N - mu * mu
    inv = lax.rsqrt(var + 1e-5)
    xb = g_ref[...] * (o_ref[...] - mu) * inv + b_ref[...]
    el = jnp.where(xb > 0, xb, jnp.exp(xb) - 1.0)
    y = el + res_ref[...]
    y = jnp.maximum(jnp.dot(y, w0_ref[...], preferred_element_type=jnp.float32)
                    + b0_ref[...], 0.0)
    y = jnp.maximum(jnp.dot(y, w1_ref[...], preferred_element_type=jnp.float32)
                    + b1_ref[...], 0.0)
    y_ref[...] = jnp.dot(y, w2_ref[...], preferred_element_type=jnp.float32) + b2_ref[...]


def _bnmlp(o_raw, st, gamma, beta, res, w0, b0, w1, b1, w2, b2):
    r_spec = pl.BlockSpec((R, D), lambda i: (i, 0))
    b_spec = pl.BlockSpec((1, D), lambda i: (0, 0))
    return pl.pallas_call(
        _bnmlp_body,
        grid=(GRID,),
        in_specs=[r_spec, pl.BlockSpec((8, D), lambda i: (0, 0)),
                  b_spec, b_spec, r_spec,
                  pl.BlockSpec((D, 64), lambda i: (0, 0)),
                  pl.BlockSpec((1, 64), lambda i: (0, 0)),
                  pl.BlockSpec((64, 32), lambda i: (0, 0)),
                  pl.BlockSpec((1, 32), lambda i: (0, 0)),
                  pl.BlockSpec((32, 7), lambda i: (0, 0)),
                  pl.BlockSpec((1, 7), lambda i: (0, 0))],
        out_specs=[pl.BlockSpec((R, 7), lambda i: (i, 0))],
        out_shape=[jax.ShapeDtypeStruct((N, 7), jnp.float32)],
    )(o_raw, st, gamma.reshape(1, D), beta.reshape(1, D), res,
      w0, b0.reshape(1, 64), w1, b1.reshape(1, 32), w2, b2.reshape(1, 7))[0]


_MESH = plsc.VectorSubcoreMesh(core_axis_name="c", subcore_axis_name="s")


def _offset_idx(dst_buf, src_buf, n, off):
    for k in range(n // 16):
        v = src_buf[0, pl.ds(16 * k, 16)]
        dst_buf[0, pl.ds(16 * k, 16)] = v + off


def _attn_math(re, fs_rows, fd_rows, o_rows, attn_v):
    for j in range(4):
        av = attn_v[j, :]
        acc = None
        for d in range(16):
            col = jnp.full((16,), j * 16 + d, jnp.int32)
            f = plsc.load_gather(fs_rows, [re, col])
            g = plsc.load_gather(fd_rows, [re, col])
            z = f + g
            z = jnp.where(z >= 0.0, z, z * NEG)
            p = z * av[d]
            acc = p if acc is None else acc + p
        ex = jnp.exp(acc)
        plsc.store_scatter(o_rows, [re, jnp.full((16,), HW + j, jnp.int32)], ex)
        for d in range(16):
            col = jnp.full((16,), j * 16 + d, jnp.int32)
            f = plsc.load_gather(fs_rows, [re, col])
            plsc.store_scatter(o_rows, [re, col], ex * f)


def _logit_math(re, gi, fs_rows, fd_rows, l_buf, attn_v):
    acc = None
    for j in range(4):
        av = attn_v[j, :]
        for d in range(16):
            col = jnp.full((16,), j * 16 + d, jnp.int32)
            f = plsc.load_gather(fs_rows, [re, col])
            g = plsc.load_gather(fd_rows, [re, col])
            z = f + g
            z = jnp.where(z >= 0.0, z, z * NEG)
            p = z * av[d]
            acc = p if acc is None else acc + p
    l_buf[0, pl.ds(gi * 16, 16)] = acc


def _weight_math(re, gi, fs_rows, ex_buf, o_rows):
    exv = ex_buf[0, pl.ds(gi * 16, 16)]
    plsc.store_scatter(o_rows, [re, jnp.full((16,), HW, jnp.int32)], exv)
    for d in range(HW):
        col = jnp.full((16,), d, jnp.int32)
        f = plsc.load_gather(fs_rows, [re, col])
        plsc.store_scatter(o_rows, [re, col], exv * f)


def _zero_acc(z_hbm, o_rows, acc, r0):
    pltpu.sync_copy(z_hbm, o_rows)
    for k in range(RPT // CH):
        pltpu.sync_copy(o_rows, acc.at[pl.ds(r0 + k * CH, CH)])


def _copy_out(acc, o_rows, out_hbm, cid, r0):
    for k in range(RPT // CH):
        pltpu.sync_copy(acc.at[pl.ds(r0 + k * CH, CH)], o_rows)
        pltpu.sync_copy(o_rows, out_hbm.at[cid, pl.ds(r0 + k * CH, CH)])


def _sc_l1_body(fs_hbm, fd_hbm, src_hbm, dst_hbm, attn_hbm, z_hbm,
                out_hbm,
                srcb, dstb, dstgb, tsrcb, tdstb, tdstgb,
                fs_rows, fd_rows, o_rows, attn_v, acc, sem1, sem2):
    cid = lax.axis_index("c")
    sid = lax.axis_index("s")
    r0 = sid * RPT
    cn = cid * N
    _zero_acc(z_hbm, o_rows, acc, r0)
    pltpu.sync_copy(attn_hbm.at[pl.ds(cid * 4, 4)], attn_v)
    plsc.subcore_barrier()
    iota = lax.iota(jnp.int32, 16)
    base_e = sid * EPT

    @pl.loop(0, NFULL)
    def _chunk(c):
        e0 = base_e + c * CH
        pltpu.sync_copy(src_hbm.at[pl.ds(e0, CH)], srcb.at[0])
        pltpu.sync_copy(dst_hbm.at[pl.ds(e0, CH)], dstb.at[0])
        _offset_idx(srcb, srcb, CH, cn)
        _offset_idx(dstgb, dstb, CH, cn)
        cp1 = pltpu.async_copy(fs_hbm.at[srcb.at[0]], fs_rows, sem1)
        cp2 = pltpu.async_copy(fd_hbm.at[dstgb.at[0]], fd_rows, sem2)
        cp1.wait()
        cp2.wait()

        @pl.loop(0, CH // 16)
        def _grp(gi):
            _attn_math(iota + gi * 16, fs_rows, fd_rows, o_rows, attn_v)

        pltpu.sync_copy(o_rows, acc.at[dstb.at[0]], add=True)

    if TAIL:
        e0 = base_e + NFULL * CH
        pltpu.sync_copy(src_hbm.at[pl.ds(e0, TAIL)], tsrcb.at[0])
        pltpu.sync_copy(dst_hbm.at[pl.ds(e0, TAIL)], tdstb.at[0])
        _offset_idx(tsrcb, tsrcb, TAIL, cn)
        _offset_idx(tdstgb, tdstb, TAIL, cn)
        cp1 = pltpu.async_copy(fs_hbm.at[tsrcb.at[0]],
                               fs_rows.at[pl.ds(0, TAIL)], sem1)
        cp2 = pltpu.async_copy(fd_hbm.at[tdstgb.at[0]],
                               fd_rows.at[pl.ds(0, TAIL)], sem2)
        cp1.wait()
        cp2.wait()

        @pl.loop(0, TAIL // 16)
        def _grp_t(gi):
            _attn_math(iota + gi * 16, fs_rows, fd_rows, o_rows, attn_v)

        pltpu.sync_copy(o_rows.at[pl.ds(0, TAIL)],
                        acc.at[tdstb.at[0]], add=True)

    plsc.subcore_barrier()
    _copy_out(acc, o_rows, out_hbm, cid, r0)


def _sc_l2a_body(fs_hbm, fd_hbm, src_hbm, dst_hbm, attn_hbm,
                 plog_hbm,
                 srcb, dstb, tsrcb, tdstb,
                 fs_rows, fd_rows, l_buf, attn_v, sem1, sem2):
    cid = lax.axis_index("c")
    sid = lax.axis_index("s")
    cn = cid * N
    pltpu.sync_copy(attn_hbm.at[pl.ds(cid * 4, 4)], attn_v)
    iota = lax.iota(jnp.int32, 16)
    base_e = sid * EPT

    @pl.loop(0, NFULL)
    def _chunk(c):
        e0 = base_e + c * CH
        pltpu.sync_copy(src_hbm.at[pl.ds(e0, CH)], srcb.at[0])
        pltpu.sync_copy(dst_hbm.at[pl.ds(e0, CH)], dstb.at[0])
        _offset_idx(srcb, srcb, CH, cn)
        _offset_idx(dstb, dstb, CH, cn)
        cp1 = pltpu.async_copy(fs_hbm.at[srcb.at[0]], fs_rows, sem1)
        cp2 = pltpu.async_copy(fd_hbm.at[dstb.at[0]], fd_rows, sem2)
        cp1.wait()
        cp2.wait()

        @pl.loop(0, CH // 16)
        def _grp(gi):
            _logit_math(iota + gi * 16, gi, fs_rows, fd_rows, l_buf, attn_v)

        pltpu.sync_copy(l_buf.at[0], plog_hbm.at[cid, pl.ds(e0, CH)])

    if TAIL:
        e0 = base_e + NFULL * CH
        pltpu.sync_copy(src_hbm.at[pl.ds(e0, TAIL)], tsrcb.at[0])
        pltpu.sync_copy(dst_hbm.at[pl.ds(e0, TAIL)], tdstb.at[0])
        _offset_idx(tsrcb, tsrcb, TAIL, cn)
        _offset_idx(tdstb, tdstb, TAIL, cn)
        cp1 = pltpu.async_copy(fs_hbm.at[tsrcb.at[0]],
                               fs_rows.at[pl.ds(0, TAIL)], sem1)
        cp2 = pltpu.async_copy(fd_hbm.at[tdstb.at[0]],
                               fd_rows.at[pl.ds(0, TAIL)], sem2)
        cp1.wait()
        cp2.wait()

        @pl.loop(0, TAIL // 16)
        def _grp_t(gi):
            _logit_math(iota + gi * 16, gi, fs_rows, fd_rows, l_buf, attn_v)

        pltpu.sync_copy(l_buf.at[0, pl.ds(0, TAIL)],
                        plog_hbm.at[cid, pl.ds(e0, TAIL)])


def _sc_l2b_body(fs_hbm, src_hbm, dst_hbm, ex_hbm, z_hbm,
                 out_hbm,
                 srcb, dstb, tsrcb, tdstb,
                 fs_rows, o_rows, ex_buf, acc, sem1, sem2):
    cid = lax.axis_index("c")
    sid = lax.axis_index("s")
    r0 = sid * RPT
    cn = cid * N
    _zero_acc(z_hbm, o_rows, acc, r0)
    plsc.subcore_barrier()
    iota = lax.iota(jnp.int32, 16)
    base_e = sid * EPT

    @pl.loop(0, NFULL)
    def _chunk(c):
        e0 = base_e + c * CH
        pltpu.sync_copy(src_hbm.at[pl.ds(e0, CH)], srcb.at[0])
        pltpu.sync_copy(dst_hbm.at[pl.ds(e0, CH)], dstb.at[0])
        pltpu.sync_copy(ex_hbm.at[pl.ds(e0, CH)], ex_buf.at[0])
        _offset_idx(srcb, srcb, CH, cn)
        cp1 = pltpu.async_copy(fs_hbm.at[srcb.at[0]], fs_rows, sem1)
        cp1.wait()

        @pl.loop(0, CH // 16)
        def _grp(gi):
            _weight_math(iota + gi * 16, gi, fs_rows, ex_buf, o_rows)

        pltpu.sync_copy(o_rows, acc.at[dstb.at[0]], add=True)

    if TAIL:
        e0 = base_e + NFULL * CH
        pltpu.sync_copy(src_hbm.at[pl.ds(e0, TAIL)], tsrcb.at[0])
        pltpu.sync_copy(dst_hbm.at[pl.ds(e0, TAIL)], tdstb.at[0])
        pltpu.sync_copy(ex_hbm.at[pl.ds(e0, TAIL)], ex_buf.at[0, pl.ds(0, TAIL)])
        _offset_idx(tsrcb, tsrcb, TAIL, cn)
        cp1 = pltpu.async_copy(fs_hbm.at[tsrcb.at[0]],
                               fs_rows.at[pl.ds(0, TAIL)], sem1)
        cp1.wait()

        @pl.loop(0, TAIL // 16)
        def _grp_t(gi):
            _weight_math(iota + gi * 16, gi, fs_rows, ex_buf, o_rows)

        pltpu.sync_copy(o_rows.at[pl.ds(0, TAIL)],
                        acc.at[tdstb.at[0]], add=True)

    plsc.subcore_barrier()
    _copy_out(acc, o_rows, out_hbm, cid, r0)


_sc_l1 = pl.kernel(
    _sc_l1_body,
    out_type=[jax.ShapeDtypeStruct((2, NPAD, AW), jnp.float32)],
    mesh=_MESH,
    compiler_params=_SC_PARAMS,
    scratch_types=[
        pltpu.VMEM((1, CH), jnp.int32),
        pltpu.VMEM((1, CH), jnp.int32),
        pltpu.VMEM((1, CH), jnp.int32),
        pltpu.VMEM((1, TAIL), jnp.int32),
        pltpu.VMEM((1, TAIL), jnp.int32),
        pltpu.VMEM((1, TAIL), jnp.int32),
        pltpu.VMEM((CH, HW), jnp.float32),
        pltpu.VMEM((CH, HW), jnp.float32),
        pltpu.VMEM((CH, AW), jnp.float32),
        pltpu.VMEM((4, 16), jnp.float32),
        pltpu.VMEM_SHARED((NPAD, AW), jnp.float32),
        pltpu.SemaphoreType.DMA,
        pltpu.SemaphoreType.DMA,
    ],
)

_sc_l2a = pl.kernel(
    _sc_l2a_body,
    out_type=[jax.ShapeDtypeStruct((2, E), jnp.float32)],
    mesh=_MESH,
    compiler_params=_SC_PARAMS,
    scratch_types=[
        pltpu.VMEM((1, CH), jnp.int32),
        pltpu.VMEM((1, CH), jnp.int32),
        pltpu.VMEM((1, TAIL), jnp.int32),
        pltpu.VMEM((1, TAIL), jnp.int32),
        pltpu.VMEM((CH, HW), jnp.float32),
        pltpu.VMEM((CH, HW), jnp.float32),
        pltpu.VMEM((1, CH), jnp.float32),
        pltpu.VMEM((4, 16), jnp.float32),
        pltpu.SemaphoreType.DMA,
        pltpu.SemaphoreType.DMA,
    ],
)

_sc_l2b = pl.kernel(
    _sc_l2b_body,
    out_type=[jax.ShapeDtypeStruct((2, NPAD, AW), jnp.float32)],
    mesh=_MESH,
    compiler_params=_SC_PARAMS,
    scratch_types=[
        pltpu.VMEM((1, CH), jnp.int32),
        pltpu.VMEM((1, CH), jnp.int32),
        pltpu.VMEM((1, TAIL), jnp.int32),
        pltpu.VMEM((1, TAIL), jnp.int32),
        pltpu.VMEM((CH, HW), jnp.float32),
        pltpu.VMEM((CH, AW), jnp.float32),
        pltpu.VMEM((1, CH), jnp.float32),
        pltpu.VMEM_SHARED((NPAD, AW), jnp.float32),
        pltpu.SemaphoreType.DMA,
        pltpu.SemaphoreType.DMA,
    ],
)


def kernel(g, h, e, W_feat, b_feat, Wl1, bl1, Wr1, br1, attn1, gamma1, beta1,
           Wl2, bl2, Wr2, br2, attn2, gamma2, beta2, Wm0, bm0, Wm1, bm1,
           Wm2, bm2):
    src = g[0]
    dst = g[1]
    z80 = jnp.zeros((CH, AW), jnp.float32)
    expand = jnp.repeat(jnp.eye(8, dtype=jnp.float32), 16, axis=1)

    x, fsA, fsB, fdA, fdB = _proj3(h, W_feat, b_feat, Wl1, bl1, Wr1, br1)
    fs_st = jnp.concatenate([fsA, fsB], axis=0)
    fd_st = jnp.concatenate([fdA, fdB], axis=0)
    (out_p,) = _sc_l1(fs_st, fd_st, src, dst, attn1, z80)
    o1, st1 = _combine(8, out_p[:, :N], expand)

    x2, fs2A, fs2B, fd2A, fd2B = _bnproj(o1, st1, gamma1, beta1, x,
                                         Wl2, bl2, Wr2, br2)
    fs2_st = jnp.concatenate([fs2A, fs2B], axis=0)
    fd2_st = jnp.concatenate([fd2A, fd2B], axis=0)
    (plog,) = _sc_l2a(fs2_st, fd2_st, src, dst, attn2.reshape(8, 16))
    ex = _edge_exp(plog.reshape(2, E // D, D)).reshape(E)
    (out_p2,) = _sc_l2b(fs2_st, src, dst, ex, z80)
    o2, st2 = _combine(1, out_p2[:, :N], expand)
    return _bnmlp(o2, st2, gamma2, beta2, x2, Wm0, bm0, Wm1, bm1, Wm2, bm2)


# speedup vs baseline: 12.2299x; 1.6771x over previous
import functools

import jax
import jax.numpy as jnp
from jax import lax
from jax.experimental import pallas as pl
from jax.experimental.pallas import tpu as pltpu
from jax.experimental.pallas import tpu_sc as plsc

N = 10000
E = 320000
D = 128
HW = 64
AW = 80
NEG = 0.2

SC_TILES = 16
EPT = E // SC_TILES
CH = 128
NFULL = EPT // CH
TAIL = EPT - NFULL * CH
NPAD = 10240
RPT = NPAD // SC_TILES

R = 1000
GRID = N // R
EB = E // GRID

_SC_PARAMS = pltpu.CompilerParams(needs_layout_passes=False,
                                  use_tc_tiling_on_sc=False)


def _proj3_body(h_ref, wf_ref, bf_ref, wl_ref, bl_ref, wr_ref, br_ref,
                x_ref, fsa_ref, fsb_ref, fda_ref, fdb_ref):
    x = jnp.dot(h_ref[...], wf_ref[...], preferred_element_type=jnp.float32)
    x = x + bf_ref[...]
    x_ref[...] = x
    fs = jnp.dot(x, wl_ref[...], preferred_element_type=jnp.float32) + bl_ref[...]
    fd = jnp.dot(x, wr_ref[...], preferred_element_type=jnp.float32) + br_ref[...]
    fsa_ref[...] = fs[:, :HW]
    fsb_ref[...] = fs[:, HW:]
    fda_ref[...] = fd[:, :HW]
    fdb_ref[...] = fd[:, HW:]


def _proj3(h, wf, bf, wl, bl, wr, br):
    w_spec = pl.BlockSpec((D, D), lambda i: (0, 0))
    b_spec = pl.BlockSpec((1, D), lambda i: (0, 0))
    r_spec = pl.BlockSpec((R, D), lambda i: (i, 0))
    h_spec = pl.BlockSpec((R, HW), lambda i: (i, 0))
    return pl.pallas_call(
        _proj3_body,
        grid=(GRID,),
        in_specs=[r_spec, w_spec, b_spec, w_spec, b_spec, w_spec, b_spec],
        out_specs=[r_spec, h_spec, h_spec, h_spec, h_spec],
        out_shape=[jax.ShapeDtypeStruct((N, D), jnp.float32)]
        + [jax.ShapeDtypeStruct((N, HW), jnp.float32)] * 4,
    )(h, wf, bf.reshape(1, D), wl, bl.reshape(1, D), wr, br.reshape(1, D))


def _combine_body(heads, op_ref, exp_ref, o_ref, st_ref):
    i = pl.program_id(0)
    o = jnp.concatenate([op_ref[0, :, :HW], op_ref[1, :, :HW]], axis=1)
    if heads == 8:
        s8 = jnp.concatenate([op_ref[0, :, HW:HW + 4],
                              op_ref[1, :, HW:HW + 4]], axis=1)
        den = jnp.dot(s8, exp_ref[...], preferred_element_type=jnp.float32) + 1e-9
        o = o / den
    else:
        o = o / (op_ref[0, :, HW:HW + 1] + 1e-9)
    o_ref[...] = o

    @pl.when(i == 0)
    def _():
        st_ref[...] = jnp.zeros_like(st_ref)

    upd = jnp.concatenate(
        [jnp.sum(o, axis=0, keepdims=True),
         jnp.sum(o * o, axis=0, keepdims=True),
         jnp.zeros((6, D), jnp.float32)], axis=0)
    st_ref[...] = st_ref[...] + upd


def _combine(heads, out_p, expand):
    return pl.pallas_call(
        functools.partial(_combine_body, heads),
        grid=(GRID,),
        in_specs=[pl.BlockSpec((2, R, AW), lambda i: (0, i, 0)),
                  pl.BlockSpec((8, D), lambda i: (0, 0))],
        out_specs=[pl.BlockSpec((R, D), lambda i: (i, 0)),
                   pl.BlockSpec((8, D), lambda i: (0, 0))],
        out_shape=[jax.ShapeDtypeStruct((N, D), jnp.float32),
                   jax.ShapeDtypeStruct((8, D), jnp.float32)],
    )(out_p, expand)


def _bnproj_body(o_ref, st_ref, g_ref, b_ref, res_ref, wl_ref, bl_ref,
                 wr_ref, br_ref, x2_ref, fsa_ref, fsb_ref, fda_ref, fdb_ref):
    mu = st_ref[0:1, :] / N
    var = st_ref[1:2, :] / N - mu * mu
    inv = lax.rsqrt(var + 1e-5)
    xb = g_ref[...] * (o_ref[...] - mu) * inv + b_ref[...]
    el = jnp.where(xb > 0, xb, jnp.exp(xb) - 1.0)
    x2 = el + res_ref[...]
    x2_ref[...] = x2
    fs = jnp.dot(x2, wl_ref[...], preferred_element_type=jnp.float32) + bl_ref[...]
    fd = jnp.dot(x2, wr_ref[...], preferred_element_type=jnp.float32) + br_ref[...]
    fsa_ref[...] = fs[:, :HW]
    fsb_ref[...] = fs[:, HW:]
    fda_ref[...] = fd[:, :HW]
    fdb_ref[...] = fd[:, HW:]


def _bnproj(o_raw, st, gamma, beta, res, wl, bl, wr, br):
    w_spec = pl.BlockSpec((D, D), lambda i: (0, 0))
    b_spec = pl.BlockSpec((1, D), lambda i: (0, 0))
    r_spec = pl.BlockSpec((R, D), lambda i: (i, 0))
    h_spec = pl.BlockSpec((R, HW), lambda i: (i, 0))
    return pl.pallas_call(
        _bnproj_body,
        grid=(GRID,),
        in_specs=[r_spec, pl.BlockSpec((8, D), lambda i: (0, 0)),
                  b_spec, b_spec, r_spec, w_spec, b_spec, w_spec, b_spec],
        out_specs=[r_spec, h_spec, h_spec, h_spec, h_spec],
        out_shape=[jax.ShapeDtypeStruct((N, D), jnp.float32)]
        + [jax.ShapeDtypeStruct((N, HW), jnp.float32)] * 4,
    )(o_raw, st, gamma.reshape(1, D), beta.reshape(1, D), res,
      wl, bl.reshape(1, D), wr, br.reshape(1, D))


def _edge_exp_body(p_ref, ex_ref):
    ex_ref[...] = jnp.exp(p_ref[0] + p_ref[1])


def _edge_exp(plog3):
    nb = E // D
    return pl.pallas_call(
        _edge_exp_body,
        in_specs=[pl.BlockSpec((2, nb, D), lambda: (0, 0, 0))],
        out_specs=pl.BlockSpec((nb, D), lambda: (0, 0)),
        out_shape=jax.ShapeDtypeStruct((nb, D), jnp.float32),
    )(plog3)


def _bnmlp_body(o_ref, st_ref, g_ref, b_ref, res_ref, w0_ref, b0_ref,
                w1_ref, b1_ref, w2_ref, b2_ref, y_ref):
    mu = st_ref[0:1, :] / N
    var = st_ref[1:2, :] / N - mu * mu
    inv = lax.rsqrt(var + 1e-5)
    xb = g_ref[...] * (o_ref[...] - mu) * inv + b_ref[...]
    el = jnp.where(xb > 0, xb, jnp.exp(xb) - 1.0)
    y = el + res_ref[...]
    y = jnp.maximum(jnp.dot(y, w0_ref[...], preferred_element_type=jnp.float32)
                    + b0_ref[...], 0.0)
    y = jnp.maximum(jnp.dot(y, w1_ref[...], preferred_element_type=jnp.float32)
                    + b1_ref[...], 0.0)
    y_ref[...] = jnp.dot(y, w2_ref[...], preferred_element_type=jnp.float32) + b2_ref[...]


def _bnmlp(o_raw, st, gamma, beta, res, w0, b0, w1, b1, w2, b2):
    r_spec = pl.BlockSpec((R, D), lambda i: (i, 0))
    b_spec = pl.BlockSpec((1, D), lambda i: (0, 0))
    return pl.pallas_call(
        _bnmlp_body,
        grid=(GRID,),
        in_specs=[r_spec, pl.BlockSpec((8, D), lambda i: (0, 0)),
                  b_spec, b_spec, r_spec,
                  pl.BlockSpec((D, 64), lambda i: (0, 0)),
                  pl.BlockSpec((1, 64), lambda i: (0, 0)),
                  pl.BlockSpec((64, 32), lambda i: (0, 0)),
                  pl.BlockSpec((1, 32), lambda i: (0, 0)),
                  pl.BlockSpec((32, 7), lambda i: (0, 0)),
                  pl.BlockSpec((1, 7), lambda i: (0, 0))],
        out_specs=[pl.BlockSpec((R, 7), lambda i: (i, 0))],
        out_shape=[jax.ShapeDtypeStruct((N, 7), jnp.float32)],
    )(o_raw, st, gamma.reshape(1, D), beta.reshape(1, D), res,
      w0, b0.reshape(1, 64), w1, b1.reshape(1, 32), w2, b2.reshape(1, 7))[0]


_MESH = plsc.VectorSubcoreMesh(core_axis_name="c", subcore_axis_name="s")

SU = 12
ROWS = E // CH
RPTILE = 156
NSUPER = RPTILE // SU
EXTRA0 = SC_TILES * RPTILE
NEXTRA = ROWS - EXTRA0
DEPTH = 3


def _offset_rows(dst_buf, src_buf, nrows, off):
    for k in range(nrows):
        for q in range(CH // 16):
            v = src_buf[k, pl.ds(16 * q, 16)]
            dst_buf[k, pl.ds(16 * q, 16)] = v + off


def _attn_math(re, fs_rows, fd_rows, o_rows, attn_v):
    for j in range(4):
        av = attn_v[j, :]
        acc = None
        fsave = []
        for d in range(16):
            col = jnp.full((16,), j * 16 + d, jnp.int32)
            f = plsc.load_gather(fs_rows, [re, col])
            g = plsc.load_gather(fd_rows, [re, col])
            fsave.append(f)
            z = f + g
            z = jnp.where(z >= 0.0, z, z * NEG)
            p = z * av[d]
            acc = p if acc is None else acc + p
        ex = jnp.exp(acc)
        plsc.store_scatter(o_rows, [re, jnp.full((16,), HW + j, jnp.int32)], ex)
        for d in range(16):
            col = jnp.full((16,), j * 16 + d, jnp.int32)
            plsc.store_scatter(o_rows, [re, col], ex * fsave[d])


def _logit_math(re, gi, fs_rows, fd_rows, l_row, attn_v):
    acc = None
    for j in range(4):
        av = attn_v[j, :]
        for d in range(16):
            col = jnp.full((16,), j * 16 + d, jnp.int32)
            f = plsc.load_gather(fs_rows, [re, col])
            g = plsc.load_gather(fd_rows, [re, col])
            z = f + g
            z = jnp.where(z >= 0.0, z, z * NEG)
            p = z * av[d]
            acc = p if acc is None else acc + p
    l_row[pl.ds(gi * 16, 16)] = acc


def _weight_math(re, gi, fs_rows, ex_row, o_rows):
    exv = ex_row[pl.ds(gi * 16, 16)]
    plsc.store_scatter(o_rows, [re, jnp.full((16,), HW, jnp.int32)], exv)
    for d in range(HW):
        col = jnp.full((16,), d, jnp.int32)
        f = plsc.load_gather(fs_rows, [re, col])
        plsc.store_scatter(o_rows, [re, col], exv * f)


def _zero_acc(z_hbm, o_buf, acc, r0):
    pltpu.sync_copy(z_hbm, o_buf)
    for k in range(RPT // CH):
        pltpu.sync_copy(o_buf, acc.at[pl.ds(r0 + k * CH, CH)])


def _copy_out(acc, o_buf, out_hbm, cid, r0):
    for k in range(RPT // CH):
        pltpu.sync_copy(acc.at[pl.ds(r0 + k * CH, CH)], o_buf)
        pltpu.sync_copy(o_buf, out_hbm.at[cid, pl.ds(r0 + k * CH, CH)])


def _make_sc_body(kind):
    use_fd = kind in ("l1", "l2a")

    def body(*refs):
        if kind == "l1":
            (fs_hbm, fd_hbm, src_hbm, dst_hbm, attn_hbm, z_hbm, out_hbm,
             sidx, didx, didxg, fs0, fs1, fs2, fd0, fd1, fd2, o0, o1, attn_v, acc,
             semf0, semf1, semf2, semg0, semg1, semg2,
             semsc0, semsc1) = refs
        elif kind == "l2a":
            (fs_hbm, fd_hbm, src_hbm, dst_hbm, attn_hbm, plog_hbm,
             sidx, didx, didxg, fs0, fs1, fs2, fd0, fd1, fd2, l_buf, attn_v,
             semf0, semf1, semf2, semg0, semg1, semg2) = refs
        else:
            (fs_hbm, src_hbm, dst_hbm, ex_hbm, z_hbm, out_hbm,
             sidx, didx, ex_buf, fs0, fs1, fs2, o0, o1, acc,
             semf0, semf1, semf2, semsc0, semsc1) = refs

        cid = lax.axis_index("c")
        sid = lax.axis_index("s")
        cn = cid * N
        r0 = sid * RPT
        fsb = [fs0, fs1, fs2]
        semf = [semf0, semf1, semf2]
        if use_fd:
            fdb = [fd0, fd1, fd2]
            semg = [semg0, semg1, semg2]
        if kind != "l2a":
            ob = [o0, o1]
            semsc = [semsc0, semsc1]
            _zero_acc(z_hbm, o0, acc, r0)
            pltpu.sync_copy(z_hbm, o1)
        if kind in ("l1", "l2a"):
            pltpu.sync_copy(attn_hbm.at[pl.ds(cid * 4, 4)], attn_v)
        if kind != "l2a":
            plsc.subcore_barrier()
        iota = lax.iota(jnp.int32, 16)
        base_row = sid * RPTILE

        def do_super(row0, nrows):
            pltpu.sync_copy(src_hbm.at[pl.ds(row0, nrows)],
                            sidx.at[pl.ds(0, nrows)])
            pltpu.sync_copy(dst_hbm.at[pl.ds(row0, nrows)],
                            didx.at[pl.ds(0, nrows)])
            _offset_rows(sidx, sidx, nrows, cn)
            if use_fd:
                _offset_rows(didxg, didx, nrows, cn)
            if kind == "l2b":
                pltpu.sync_copy(ex_hbm.at[pl.ds(row0, nrows)],
                                ex_buf.at[pl.ds(0, nrows)])

            def issue(k):
                p = k % DEPTH
                cpf = pltpu.async_copy(fs_hbm.at[sidx.at[k]], fsb[p], semf[p])
                cpg = (pltpu.async_copy(fd_hbm.at[didxg.at[k]], fdb[p], semg[p])
                       if use_fd else None)
                return cpf, cpg

            pend = {k: issue(k) for k in range(min(DEPTH - 1, nrows))}
            sc_pend = [None, None]
            for k in range(nrows):
                p = k % DEPTH
                po = k % 2
                if k + DEPTH - 1 < nrows:
                    pend[k + DEPTH - 1] = issue(k + DEPTH - 1)
                pend[k][0].wait()
                if use_fd:
                    pend[k][1].wait()
                del pend[k]
                if sc_pend[po] is not None:
                    sc_pend[po].wait()

                if kind == "l1":
                    @pl.loop(0, CH // 16)
                    def _grp(gi):
                        _attn_math(iota + gi * 16, fsb[p], fdb[p], ob[po],
                                   attn_v)
                    sc_pend[po] = pltpu.async_copy(
                        ob[po], acc.at[didx.at[k]], semsc[po], add=True)
                elif kind == "l2a":
                    @pl.loop(0, CH // 16)
                    def _grp(gi):
                        _logit_math(iota + gi * 16, gi, fsb[p], fdb[p],
                                    l_buf.at[k], attn_v)
                else:
                    @pl.loop(0, CH // 16)
                    def _grp(gi):
                        _weight_math(iota + gi * 16, gi, fsb[p],
                                     ex_buf.at[k], ob[po])
                    sc_pend[po] = pltpu.async_copy(
                        ob[po], acc.at[didx.at[k]], semsc[po], add=True)

            for p in range(2):
                if sc_pend[p] is not None:
                    sc_pend[p].wait()
            if kind == "l2a":
                pltpu.sync_copy(l_buf.at[pl.ds(0, nrows)],
                                plog_hbm.at[cid, pl.ds(row0, nrows)])

        @pl.loop(0, NSUPER)
        def _super(su):
            do_super(base_row + su * SU, SU)

        @pl.when(sid < NEXTRA)
        def _extra():
            do_super(EXTRA0 + sid, 1)

        if kind != "l2a":
            plsc.subcore_barrier()
            _copy_out(acc, o0, out_hbm, cid, r0)

    return body


_IDX2 = pltpu.VMEM((SU, CH), jnp.int32)
_FBUF = pltpu.VMEM((CH, HW), jnp.float32)
_OBUF = pltpu.VMEM((CH, AW), jnp.float32)
_SEM = pltpu.SemaphoreType.DMA

_sc_l1 = pl.kernel(
    _make_sc_body("l1"),
    out_type=[jax.ShapeDtypeStruct((2, NPAD, AW), jnp.float32)],
    mesh=_MESH,
    compiler_params=_SC_PARAMS,
    scratch_types=[
        _IDX2, _IDX2, _IDX2,
        _FBUF, _FBUF, _FBUF, _FBUF, _FBUF, _FBUF,
        _OBUF, _OBUF,
        pltpu.VMEM((4, 16), jnp.float32),
        pltpu.VMEM_SHARED((NPAD, AW), jnp.float32),
        _SEM, _SEM, _SEM, _SEM, _SEM, _SEM, _SEM, _SEM,
    ],
)

_sc_l2a = pl.kernel(
    _make_sc_body("l2a"),
    out_type=[jax.ShapeDtypeStruct((2, ROWS, CH), jnp.float32)],
    mesh=_MESH,
    compiler_params=_SC_PARAMS,
    scratch_types=[
        _IDX2, _IDX2, _IDX2,
        _FBUF, _FBUF, _FBUF, _FBUF, _FBUF, _FBUF,
        pltpu.VMEM((SU, CH), jnp.float32),
        pltpu.VMEM((4, 16), jnp.float32),
        _SEM, _SEM, _SEM, _SEM, _SEM, _SEM,
    ],
)

_sc_l2b = pl.kernel(
    _make_sc_body("l2b"),
    out_type=[jax.ShapeDtypeStruct((2, NPAD, AW), jnp.float32)],
    mesh=_MESH,
    compiler_params=_SC_PARAMS,
    scratch_types=[
        _IDX2, _IDX2,
        pltpu.VMEM((SU, CH), jnp.float32),
        _FBUF, _FBUF, _FBUF,
        _OBUF, _OBUF,
        pltpu.VMEM_SHARED((NPAD, AW), jnp.float32),
        _SEM, _SEM, _SEM, _SEM, _SEM,
    ],
)


def kernel(g, h, e, W_feat, b_feat, Wl1, bl1, Wr1, br1, attn1, gamma1, beta1,
           Wl2, bl2, Wr2, br2, attn2, gamma2, beta2, Wm0, bm0, Wm1, bm1,
           Wm2, bm2):
    src2 = g[0].reshape(ROWS, CH)
    dst2 = g[1].reshape(ROWS, CH)
    z80 = jnp.zeros((CH, AW), jnp.float32)
    expand = jnp.repeat(jnp.eye(8, dtype=jnp.float32), 16, axis=1)

    x, fsA, fsB, fdA, fdB = _proj3(h, W_feat, b_feat, Wl1, bl1, Wr1, br1)
    fs_st = jnp.concatenate([fsA, fsB], axis=0)
    fd_st = jnp.concatenate([fdA, fdB], axis=0)
    (out_p,) = _sc_l1(fs_st, fd_st, src2, dst2, attn1, z80)
    o1, st1 = _combine(8, out_p[:, :N], expand)

    x2, fs2A, fs2B, fd2A, fd2B = _bnproj(o1, st1, gamma1, beta1, x,
                                         Wl2, bl2, Wr2, br2)
    fs2_st = jnp.concatenate([fs2A, fs2B], axis=0)
    fd2_st = jnp.concatenate([fd2A, fd2B], axis=0)
    (plog,) = _sc_l2a(fs2_st, fd2_st, src2, dst2, attn2.reshape(8, 16))
    ex2 = _edge_exp(plog)
    (out_p2,) = _sc_l2b(fs2_st, src2, dst2, ex2, z80)
    o2, st2 = _combine(1, out_p2[:, :N], expand)
    return _bnmlp(o2, st2, gamma2, beta2, x2, Wm0, bm0, Wm1, bm1, Wm2, bm2)
